# Initial kernel scaffold; baseline (speedup 1.0000x reference)
import jax
import jax.numpy as jnp
from jax.experimental import pallas as pl


def kernel(queries, Wq, bq, Wk, bk, Wv, bv, Wo, bo):
    raise NotImplementedError("write your pallas kernel here")


# speedup vs baseline: 8.5169x; 8.5169x over previous
import functools
import math

import numpy as np
import jax
import jax.numpy as jnp
from jax.experimental import pallas as pl
from jax.experimental.pallas import tpu as pltpu

L = 4096
D = 512
U = 45
UPAD = 64
BLK = 512
NB = L // BLK

_consts = {}


def _cnt_matrix() -> np.ndarray:
    if "cnt" not in _consts:
        with jax.ensure_compile_time_eval():
            idx = np.asarray(jax.random.randint(jax.random.key(42), (L, U), 0, L))
        cnt = np.zeros((L, L), np.int8)
        np.add.at(cnt, (np.arange(L)[:, None], idx), 1)
        _consts["cnt"] = cnt
    return _consts["cnt"]


def _tril_matrix() -> np.ndarray:
    if "tril" not in _consts:
        _consts["tril"] = np.tril(np.ones((BLK, BLK), np.float32))
    return _consts["tril"]


def _qkv_body(x_ref, wq_ref, wk_ref, wv_ref, b_ref, q_ref, k_ref, v_ref):
    x = x_ref[...]
    q_ref[...] = jnp.dot(x, wq_ref[...], preferred_element_type=jnp.float32) + b_ref[0:1, :]
    k_ref[...] = jnp.dot(x, wk_ref[...], preferred_element_type=jnp.float32) + b_ref[1:2, :]
    v_ref[...] = jnp.dot(x, wv_ref[...], preferred_element_type=jnp.float32) + b_ref[2:3, :]


def _m_body(q_ref, k_ref, cnt_ref, m_ref):
    q = q_ref[...]
    k = k_ref[...]
    s = jax.lax.dot_general(q, k, (((1,), (1,)), ((), ())),
                            preferred_element_type=jnp.float32)
    cnt = cnt_ref[...].astype(jnp.float32)
    smax = jnp.max(jnp.where(cnt > 0.0, s, -jnp.inf), axis=1)
    ssum = jnp.sum(cnt * s, axis=1)
    m_ref[...] = (smax - ssum * (1.0 / L))[None, None, :]


def _topk_body(m_ref, row_ref, col_ref):
    m = m_ref[...]
    colid = jax.lax.broadcasted_iota(jnp.int32, (1, L), 1)
    lane = jax.lax.broadcasted_iota(jnp.int32, (1, UPAD), 1)
    sub = jax.lax.broadcasted_iota(jnp.int32, (UPAD, 1), 0)

    def step(t, carry):
        m, orow, ocol = carry
        mx = jnp.max(m)
        idx = jnp.min(jnp.where(m == mx, colid, L))
        m = jnp.where(colid == idx, -jnp.inf, m)
        orow = jnp.where(lane == t, idx, orow)
        ocol = jnp.where(sub == t, idx, ocol)
        return m, orow, ocol

    _, orow, ocol = jax.lax.fori_loop(
        0, U, step,
        (m, jnp.zeros((1, UPAD), jnp.int32), jnp.zeros((UPAD, 1), jnp.int32)))
    row_ref[...] = orow
    col_ref[...] = ocol


def _gather_body(mtop_ref, q_ref, out_ref):
    out_ref[...] = q_ref[...]


def _attn_body(qs_ref, k_ref, v_ref, mcol_ref, upd_ref):
    qs = qs_ref[...]
    k = k_ref[...]
    s = jax.lax.dot_general(qs, k, (((1,), (1,)), ((), ())),
                            preferred_element_type=jnp.float32)
    s = s * (1.0 / math.sqrt(D))
    colid = jax.lax.broadcasted_iota(jnp.int32, (UPAD, L), 1)
    s = jnp.where(colid > mcol_ref[...], -jnp.inf, s)
    mx = jnp.max(s, axis=1, keepdims=True)
    p = jnp.exp(s - mx)
    attn = p / jnp.sum(p, axis=1, keepdims=True)
    upd_ref[...] = jnp.dot(attn, v_ref[...], preferred_element_type=jnp.float32)


def _ctx_body(v_ref, tril_ref, mrow_ref, upd_ref, wot_ref, bo_ref, out_ref, carry_ref):
    i = pl.program_id(0)

    @pl.when(i == 0)
    def _():
        carry_ref[...] = jnp.zeros_like(carry_ref)

    v = v_ref[...]
    ctx = jax.lax.dot_general(tril_ref[...], v, (((1,), (0,)), ((), ())),
                              preferred_element_type=jnp.float32,
                              precision=jax.lax.Precision.HIGHEST)
    ctx = ctx + carry_ref[...]
    carry_ref[...] = carry_ref[...] + jnp.sum(v, axis=0, keepdims=True)

    rowid = jax.lax.broadcasted_iota(jnp.int32, (BLK, UPAD), 0) + i * BLK
    tid = jax.lax.broadcasted_iota(jnp.int32, (BLK, UPAD), 1)
    p = jnp.logical_and(rowid == mrow_ref[...], tid < U).astype(jnp.float32)
    sel = jnp.dot(p, upd_ref[...], preferred_element_type=jnp.float32)
    hit = jnp.sum(p, axis=1, keepdims=True) > 0.0
    ctx = jnp.where(hit, sel, ctx)

    out_ref[...] = jnp.dot(ctx, wot_ref[...], preferred_element_type=jnp.float32) + bo_ref[...]


def _build(interpret: bool = False):
    call = functools.partial(pl.pallas_call, interpret=interpret)

    qkv = call(
        _qkv_body,
        grid=(NB,),
        in_specs=[
            pl.BlockSpec((BLK, D), lambda i: (i, 0)),
            pl.BlockSpec((D, D), lambda i: (0, 0)),
            pl.BlockSpec((D, D), lambda i: (0, 0)),
            pl.BlockSpec((D, D), lambda i: (0, 0)),
            pl.BlockSpec((3, D), lambda i: (0, 0)),
        ],
        out_specs=[
            pl.BlockSpec((BLK, D), lambda i: (i, 0)),
            pl.BlockSpec((BLK, D), lambda i: (i, 0)),
            pl.BlockSpec((BLK, D), lambda i: (i, 0)),
        ],
        out_shape=[jax.ShapeDtypeStruct((L, D), jnp.float32)] * 3,
    )

    mst = call(
        _m_body,
        grid=(NB,),
        in_specs=[
            pl.BlockSpec((BLK, D), lambda i: (i, 0)),
            pl.BlockSpec((L, D), lambda i: (0, 0)),
            pl.BlockSpec((BLK, L), lambda i: (i, 0)),
        ],
        out_specs=pl.BlockSpec((1, 1, BLK), lambda i: (i, 0, 0)),
        out_shape=jax.ShapeDtypeStruct((NB, 1, BLK), jnp.float32),
    )

    topk = call(
        _topk_body,
        in_specs=[pl.BlockSpec((1, L), lambda: (0, 0))],
        out_specs=[
            pl.BlockSpec((1, UPAD), lambda: (0, 0)),
            pl.BlockSpec((UPAD, 1), lambda: (0, 0)),
        ],
        out_shape=[
            jax.ShapeDtypeStruct((1, UPAD), jnp.int32),
            jax.ShapeDtypeStruct((UPAD, 1), jnp.int32),
        ],
    )

    gather = call(
        _gather_body,
        grid_spec=pltpu.PrefetchScalarGridSpec(
            num_scalar_prefetch=1,
            grid=(UPAD,),
            in_specs=[pl.BlockSpec((1, 1, D), lambda t, m: (m[t], 0, 0))],
            out_specs=pl.BlockSpec((1, 1, D), lambda t, m: (t, 0, 0)),
        ),
        out_shape=jax.ShapeDtypeStruct((UPAD, 1, D), jnp.float32),
    )

    attn = call(
        _attn_body,
        in_specs=[
            pl.BlockSpec((UPAD, D), lambda: (0, 0)),
            pl.BlockSpec((L, D), lambda: (0, 0)),
            pl.BlockSpec((L, D), lambda: (0, 0)),
            pl.BlockSpec((UPAD, 1), lambda: (0, 0)),
        ],
        out_specs=pl.BlockSpec((UPAD, D), lambda: (0, 0)),
        out_shape=jax.ShapeDtypeStruct((UPAD, D), jnp.float32),
    )

    ctx = call(
        _ctx_body,
        grid=(NB,),
        in_specs=[
            pl.BlockSpec((BLK, D), lambda i: (i, 0)),
            pl.BlockSpec((BLK, BLK), lambda i: (0, 0)),
            pl.BlockSpec((1, UPAD), lambda i: (0, 0)),
            pl.BlockSpec((UPAD, D), lambda i: (0, 0)),
            pl.BlockSpec((D, D), lambda i: (0, 0)),
            pl.BlockSpec((1, D), lambda i: (0, 0)),
        ],
        out_specs=pl.BlockSpec((BLK, D), lambda i: (i, 0)),
        out_shape=jax.ShapeDtypeStruct((L, D), jnp.float32),
        scratch_shapes=[pltpu.VMEM((1, D), jnp.float32)],
    )

    return qkv, mst, topk, gather, attn, ctx


def _run(queries, Wq, bq, Wk, bk, Wv, bv, Wo, bo, interpret=False):
    qkv, mst, topk, gather, attn, ctx = _build(interpret)
    x = queries.reshape(L, D)
    b_all = jnp.stack([bq, bk, bv], axis=0)
    q, k, v = qkv(x, Wq.T, Wk.T, Wv.T, b_all)
    cnt = jnp.asarray(_cnt_matrix())
    m = mst(q, k, cnt)
    mrow, mcol = topk(m.reshape(1, L))
    mflat = mrow.reshape(UPAD)
    qs = gather(mflat, q.reshape(L, 1, D)).reshape(UPAD, D)
    upd = attn(qs, k, v, mcol)
    out = ctx(v, jnp.asarray(_tril_matrix()), mrow, upd, Wo.T, bo.reshape(1, D))
    return out.reshape(1, L, D)


def kernel(queries, Wq, bq, Wk, bk, Wv, bv, Wo, bo):
    return _run(queries, Wq, bq, Wk, bk, Wv, bv, Wo, bo, interpret=False)
